# Initial kernel scaffold; baseline (speedup 1.0000x reference)
import jax
import jax.numpy as jnp
from jax.experimental import pallas as pl


def kernel(input, target):
    raise NotImplementedError("write your pallas kernel here")


# speedup vs baseline: 13.0041x; 13.0041x over previous
import functools

import jax
import jax.numpy as jnp
from jax.experimental import pallas as pl
from jax.experimental.pallas import tpu as pltpu

BINS = 10
B = 16384
C = 1000
BLK = 256


def _body(x_ref, t_ref, out_ref, acc_ref):
    i = pl.program_id(0)
    nsteps = pl.num_programs(0)

    @pl.when(i == 0)
    def _init():
        for k in range(BINS):
            acc_ref[0, k] = 0.0
            acc_ref[1, k] = 0.0

    x = x_ref[...]
    t = t_ref[...]
    g = jnp.abs(jax.nn.sigmoid(x) - t)
    binf = jnp.clip(jnp.floor(g * BINS), 0.0, BINS - 1.0)
    bce = jnp.maximum(x, 0.0) - x * t + jnp.log1p(jnp.exp(-jnp.abs(x)))
    for k in range(BINS):
        m = binf == k
        acc_ref[0, k] += jnp.sum(m.astype(jnp.float32))
        acc_ref[1, k] += jnp.sum(jnp.where(m, bce, 0.0))

    @pl.when(i == nsteps - 1)
    def _finalize():
        tot = jnp.float32(B)
        n = jnp.float32(0.0)
        s = jnp.float32(0.0)
        for k in range(BINS):
            cnt = acc_ref[0, k]
            nonempty = cnt > 0.0
            n += jnp.where(nonempty, 1.0, 0.0)
            s += jnp.where(nonempty, acc_ref[1, k] / jnp.maximum(cnt, 1.0), 0.0)
        out_ref[0, 0] = s * tot / (jnp.maximum(n, 1.0) * tot)


@jax.jit
def kernel(input, target):
    out = pl.pallas_call(
        _body,
        grid=(B // BLK,),
        in_specs=[
            pl.BlockSpec((BLK, C), lambda i: (i, 0)),
            pl.BlockSpec((BLK, C), lambda i: (i, 0)),
        ],
        out_specs=pl.BlockSpec(memory_space=pltpu.SMEM),
        out_shape=jax.ShapeDtypeStruct((1, 1), jnp.float32),
        scratch_shapes=[pltpu.SMEM((2, BINS), jnp.float32)],
    )(input, target)
    return out[0, 0]


# speedup vs baseline: 14.4200x; 1.1089x over previous
import math

import jax
import jax.numpy as jnp
from jax.experimental import pallas as pl
from jax.experimental.pallas import tpu as pltpu

BINS = 10
B = 16384
C = 1000
BLK = 256

_THRESH = [math.log(k / (BINS - k)) for k in range(1, BINS)]


def _body(x_ref, t_ref, out_ref, acc_ref):
    i = pl.program_id(0)
    nsteps = pl.num_programs(0)

    @pl.when(i == 0)
    def _init():
        for k in range(2 * BINS):
            acc_ref[k] = 0.0

    x = x_ref[...]
    t = t_ref[...]
    y = jnp.where(t > 0.5, -x, x)
    bce = jnp.maximum(y, 0.0) + jnp.log1p(jnp.exp(-jnp.abs(y)))
    acc_ref[BINS] += jnp.sum(bce)
    for k in range(1, BINS):
        m = y >= _THRESH[k - 1]
        acc_ref[k] += jnp.sum(m.astype(jnp.float32))
        acc_ref[BINS + k] += jnp.sum(jnp.where(m, bce, 0.0))

    @pl.when(i == nsteps - 1)
    def _finalize():
        n = jnp.float32(0.0)
        s = jnp.float32(0.0)
        for k in range(BINS):
            ccum_lo = jnp.float32(B * C) if k == 0 else acc_ref[k]
            ccum_hi = jnp.float32(0.0) if k == BINS - 1 else acc_ref[k + 1]
            scum_lo = acc_ref[BINS + k]
            scum_hi = jnp.float32(0.0) if k == BINS - 1 else acc_ref[BINS + k + 1]
            cnt = ccum_lo - ccum_hi
            sb = scum_lo - scum_hi
            nonempty = cnt > 0.0
            n += jnp.where(nonempty, 1.0, 0.0)
            s += jnp.where(nonempty, sb / jnp.maximum(cnt, 1.0), 0.0)
        out_ref[0] = s / jnp.maximum(n, 1.0)


@jax.jit
def kernel(input, target):
    out = pl.pallas_call(
        _body,
        grid=(B // BLK,),
        in_specs=[
            pl.BlockSpec((BLK, C), lambda i: (i, 0)),
            pl.BlockSpec((BLK, C), lambda i: (i, 0)),
        ],
        out_specs=pl.BlockSpec(memory_space=pltpu.SMEM),
        out_shape=jax.ShapeDtypeStruct((1,), jnp.float32),
        scratch_shapes=[pltpu.SMEM((2 * BINS,), jnp.float32)],
    )(input, target)
    return out[0]


# speedup vs baseline: 16.1549x; 1.1203x over previous
import math

import jax
import jax.numpy as jnp
from jax.experimental import pallas as pl
from jax.experimental.pallas import tpu as pltpu

BINS = 10
B = 16384
C = 1000
BLK = 256
RCH = 8
NACC = 2 * (BINS - 1) + 1

_THRESH = [math.log(k / (BINS - k)) for k in range(1, BINS)]

_COLS = [(ci * 128, 128) for ci in range(7)] + [(896, 104)]


def _body(x_ref, t_ref, out_ref, acc_ref):
    i = pl.program_id(0)
    nsteps = pl.num_programs(0)

    def init_accs():
        return tuple(jnp.zeros((RCH, 128), jnp.float32) for _ in range(NACC))

    def load_accs():
        return tuple(acc_ref[k] for k in range(NACC))

    accs = jax.lax.cond(i == 0, init_accs, load_accs)

    def row_chunk(r, accs):
        accs = list(accs)
        for c0, w in _COLS:
            x = x_ref[pl.ds(r * RCH, RCH), pl.ds(c0, w)]
            t = t_ref[pl.ds(r * RCH, RCH), pl.ds(c0, w)]
            if w < 128:
                x = jnp.concatenate(
                    [x, jnp.full((RCH, 128 - w), jnp.inf, jnp.float32)], axis=1)
                t = jnp.concatenate(
                    [t, jnp.ones((RCH, 128 - w), jnp.float32)], axis=1)
            y = jnp.where(t > 0.5, -x, x)
            bce = jnp.maximum(y, 0.0) + jnp.log1p(jnp.exp(-jnp.abs(y)))
            accs[0] = accs[0] + bce
            for k in range(1, BINS):
                m = y >= _THRESH[k - 1]
                accs[2 * k - 1] = accs[2 * k - 1] + m.astype(jnp.float32)
                accs[2 * k] = accs[2 * k] + jnp.where(m, bce, 0.0)
        return tuple(accs)

    accs = jax.lax.fori_loop(0, BLK // RCH, row_chunk, accs)
    for k in range(NACC):
        acc_ref[k] = accs[k]

    @pl.when(i == nsteps - 1)
    def _finalize():
        s_tot = jnp.sum(acc_ref[0])
        ccum = [jnp.float32(B * C)] + [jnp.sum(acc_ref[2 * k - 1]) for k in range(1, BINS)]
        scum = [s_tot] + [jnp.sum(acc_ref[2 * k]) for k in range(1, BINS)]
        ccum.append(jnp.float32(0.0))
        scum.append(jnp.float32(0.0))
        n = jnp.float32(0.0)
        s = jnp.float32(0.0)
        for k in range(BINS):
            cnt = ccum[k] - ccum[k + 1]
            sb = scum[k] - scum[k + 1]
            nonempty = cnt > 0.0
            n += jnp.where(nonempty, 1.0, 0.0)
            s += jnp.where(nonempty, sb / jnp.maximum(cnt, 1.0), 0.0)
        out_ref[0] = s / jnp.maximum(n, 1.0)


@jax.jit
def kernel(input, target):
    out = pl.pallas_call(
        _body,
        grid=(B // BLK,),
        in_specs=[
            pl.BlockSpec((BLK, C), lambda i: (i, 0)),
            pl.BlockSpec((BLK, C), lambda i: (i, 0)),
        ],
        out_specs=pl.BlockSpec(memory_space=pltpu.SMEM),
        out_shape=jax.ShapeDtypeStruct((1,), jnp.float32),
        scratch_shapes=[pltpu.VMEM((NACC, RCH, 128), jnp.float32)],
    )(input, target)
    return out[0]
